# baseline (speedup 1.0000x reference)
import functools

import jax
import jax.numpy as jnp
from jax import lax
from jax.experimental import pallas as pl
from jax.experimental.pallas import tpu as pltpu
from jax.experimental.pallas import tpu_sc as plsc

_L = 2048
_D = 256
_C = 256
_NCH = _L // _C
_W = 384


def _rms(x):
    return x * jax.lax.rsqrt(jnp.mean(x * x, axis=-1, keepdims=True) + 1e-6)


def _lane_iota():
    return jax.lax.broadcasted_iota(jnp.int32, (1, _L), 1).astype(jnp.float32)


def _eye_c():
    r = jax.lax.broadcasted_iota(jnp.int32, (_C, _C), 0)
    c = jax.lax.broadcasted_iota(jnp.int32, (_C, _C), 1)
    return r == c


def _col_to_row(col):
    eye = _eye_c()
    parts = []
    for c in range(_NCH):
        blk = jax.lax.slice(col, (c * _C, 0), ((c + 1) * _C, 1))
        m = jnp.where(eye, jnp.broadcast_to(blk, (_C, _C)), 0.0)
        parts.append(jnp.sum(m, axis=0, keepdims=True))
    return jnp.concatenate(parts, axis=1)


def _row_to_col_chunk(row):
    m = jnp.where(_eye_c(), jnp.broadcast_to(row, (_C, _C)), 0.0)
    return jnp.sum(m, axis=1, keepdims=True)


def _cumsum_row(x):
    s = x
    k = 1
    while k < _L:
        sh = jnp.concatenate(
            [jnp.zeros((1, k), x.dtype), jax.lax.slice(s, (0, 0), (1, _L - k))],
            axis=1)
        s = s + sh
        k *= 2
    return s


def _routing(x, m_row, Wq, bq, Wk, bk):
    dn = (((1,), (1,)), ((), ()))
    Qf = jax.lax.dot_general(x, Wq, dn, preferred_element_type=jnp.float32) + bq
    Kf = jax.lax.dot_general(x, Wk, dn, preferred_element_type=jnp.float32) + bk
    Ks = jnp.concatenate(
        [jax.lax.slice(Kf, (1, 0), (_L, _D)), jnp.zeros((1, _D), jnp.float32)],
        axis=0)
    qn = jnp.sqrt(jnp.sum(Qf * Qf, axis=1, keepdims=True))
    kn = jnp.sqrt(jnp.sum(Ks * Ks, axis=1, keepdims=True))
    Qn = Qf / jnp.maximum(qn, 1e-12)
    Kn = Ks / jnp.maximum(kn, 1e-12)
    cos = jnp.sum(Qn * Kn, axis=1, keepdims=True)
    a = jnp.clip(0.5 * (1.0 - cos), 0.0, 1.0)
    a_col = jnp.concatenate(
        [jnp.ones((1, 1), jnp.float32), jax.lax.slice(a, (0, 0), (_L - 1, 1))],
        axis=0)
    A_row = _col_to_row(a_col)
    bm_row = (A_row > 0.5) & m_row
    return A_row, bm_row


def _enc_body(h_ref, m_ref, Wq_ref, bq_ref, Wk_ref, bk_ref,
              e_ref, ord_ref, mn_ref):
    h = h_ref[0]
    m_row = m_ref[0] > 0.5
    e = _rms(h)
    e_ref[0] = e
    _, bm = _routing(e, m_row, Wq_ref[...], bq_ref[0], Wk_ref[...], bk_ref[0])
    cum = _cumsum_row(bm.astype(jnp.float32))
    num = jax.lax.slice(cum, (0, _L - 1), (1, _L))
    lane = _lane_iota()
    dest = jnp.where(bm, cum - 1.0, num + lane - cum)
    dn = (((1,), (1,)), ((), ()))
    parts = []
    for c in range(_NCH):
        ridx = (jax.lax.broadcasted_iota(jnp.int32, (_C, 1), 0).astype(jnp.float32)
                + float(c * _C))
        P = jnp.where(ridx == dest, 1.0, 0.0)
        oc = jax.lax.dot_general(P, lane, dn,
                                 preferred_element_type=jnp.float32)
        parts.append(_col_to_row_chunk_small(oc))
    ord_row = jnp.concatenate(parts, axis=1)
    ord_ref[0] = ord_row.astype(jnp.int32)
    mn_ref[0] = (lane < num).astype(jnp.float32)


def _col_to_row_chunk_small(col):
    m = jnp.where(_eye_c(), jnp.broadcast_to(col, (_C, _C)), 0.0)
    return jnp.sum(m, axis=0, keepdims=True)


def _enc_call(h, m_f, Wq, bq, Wk, bk):
    B = h.shape[0]
    return pl.pallas_call(
        _enc_body,
        grid=(B,),
        in_specs=[
            pl.BlockSpec((1, _L, _D), lambda b: (b, 0, 0)),
            pl.BlockSpec((1, 1, _L), lambda b: (b, 0, 0)),
            pl.BlockSpec((_D, _D), lambda b: (0, 0)),
            pl.BlockSpec((1, 1, _D), lambda b: (0, 0, 0)),
            pl.BlockSpec((_D, _D), lambda b: (0, 0)),
            pl.BlockSpec((1, 1, _D), lambda b: (0, 0, 0)),
        ],
        out_specs=[
            pl.BlockSpec((1, _L, _D), lambda b: (b, 0, 0)),
            pl.BlockSpec((1, 1, _L), lambda b: (b, 0, 0)),
            pl.BlockSpec((1, 1, _L), lambda b: (b, 0, 0)),
        ],
        out_shape=[
            jax.ShapeDtypeStruct((B, _L, _D), jnp.float32),
            jax.ShapeDtypeStruct((B, 1, _L), jnp.int32),
            jax.ShapeDtypeStruct((B, 1, _L), jnp.float32),
        ],
    )(h, m_f, Wq, bq.reshape(1, 1, _D), Wk, bk.reshape(1, 1, _D))


def _sc_gather(table_flat, idx2d):
    R = table_flat.shape[0]
    info = plsc.get_sparse_core_info()
    nw = info.num_cores * info.num_subcores
    rpw = R // nw
    nseg = rpw // 128
    mesh = plsc.VectorSubcoreMesh(core_axis_name="c", subcore_axis_name="s")

    @functools.partial(
        pl.kernel, mesh=mesh,
        out_type=jax.ShapeDtypeStruct((R, _D), jnp.float32),
        scratch_types=[
            pltpu.VMEM((nseg, 128), jnp.int32),
            pltpu.VMEM((rpw, _D), jnp.float32),
            pltpu.SemaphoreType.DMA,
        ],
    )
    def k(table_hbm, idx_hbm, out_hbm, idx_v, rows_v, sem):
        wid = lax.axis_index("s") * info.num_cores + lax.axis_index("c")
        pltpu.sync_copy(idx_hbm.at[pl.ds(wid * nseg, nseg)], idx_v)
        for j in range(nseg):
            pltpu.async_copy(table_hbm.at[idx_v.at[j]],
                             rows_v.at[pl.ds(j * 128, 128)], sem).wait()
        pltpu.sync_copy(rows_v, out_hbm.at[pl.ds(wid * rpw, rpw)])

    return k(table_flat, idx2d)


def _gather_rows(e, order):
    B = e.shape[0]
    offs = (jnp.arange(B, dtype=jnp.int32) * _L).reshape(B, 1, 1)
    idx_flat = (order + offs).reshape(B * _L // 128, 128)
    out = _sc_gather(e.reshape(B * _L, _D), idx_flat)
    return out.reshape(B, _L, _D)


def _upsample2(z_ref, sc_ref, z1, z2, A_row, bm_row):
    z_ref[:, 0:_D] = z1
    z_ref[:, _D:2 * _D] = z2
    p = jnp.clip(A_row, 1e-4, 1.0 - 1e-4)
    cum = _cumsum_row(bm_row.astype(jnp.float32))
    cb = jnp.maximum(cum - 1.0, 0.0)
    sc_ref[0:1, :] = cb
    S = _cumsum_row(jnp.log(1.0 - p))
    Spad = jnp.concatenate(
        [jnp.zeros((1, 1), jnp.float32), jax.lax.slice(S, (0, 0), (1, _L - 1))],
        axis=1)
    r = jax.lax.broadcasted_iota(jnp.int32, (_C, _C), 0)
    c = jax.lax.broadcasted_iota(jnp.int32, (_C, _C), 1)
    tril = r >= c
    dn = (((1,), (0,)), ((), ()))
    lane_w = jax.lax.broadcasted_iota(jnp.int32, (1, _W), 1).astype(jnp.float32)
    carry = jnp.zeros((1, 2 * _D), jnp.float32)
    outs = []
    for ch in range(_NCH):
        s0 = ch * _C
        Sp_row = jax.lax.slice(Spad, (0, s0), (1, s0 + _C))
        p_row = jax.lax.slice(p, (0, s0), (1, s0 + _C))
        cb_col = _row_to_col_chunk(jax.lax.slice(cb, (0, s0), (1, s0 + _C)))
        Sp_col = _row_to_col_chunk(Sp_row)
        base = jnp.minimum((sc_ref[0, s0].astype(jnp.int32) // 128) * 128,
                           _L - _W)
        zwin = z_ref[pl.ds(base, _W), :]
        G = jnp.where(cb_col - base.astype(jnp.float32) == lane_w, 1.0, 0.0)
        z_exp = jax.lax.dot_general(G, zwin, dn, preferred_element_type=jnp.float32)
        Wl = jnp.where(tril, p_row * jnp.exp(Sp_col - Sp_row), 0.0)
        loc = jax.lax.dot_general(Wl, z_exp, dn, preferred_element_type=jnp.float32)
        S0 = jax.lax.slice(Sp_row, (0, 0), (1, 1))
        out_c = loc + jnp.exp(Sp_col - S0) * carry
        p_last = jax.lax.slice(p_row, (0, _C - 1), (1, _C))
        out_last = jax.lax.slice(out_c, (_C - 1, 0), (_C, 2 * _D))
        carry = (1.0 - p_last) * out_last
        outs.append(out_c)
    return jnp.concatenate(outs, axis=0)


def _dec_body(h2_ref, m_ref, e1_ref, e0_ref, dWq_ref, dbq_ref, dWk_ref,
              dbk_ref, rw_ref, out_ref, z_ref, sc_ref):
    m2 = m_ref[0] > 0.5
    rw = rw_ref[0]

    d0 = _rms(h2_ref[0])
    A0, bmA = _routing(d0, m2, dWq_ref[0], dbq_ref[0], dWk_ref[0], dbk_ref[0])
    u = _upsample2(z_ref, sc_ref, d0, e1_ref[0], A0, bmA)
    h3 = (jax.lax.slice(u, (0, 0), (_L, _D))
          + jax.lax.slice(rw, (0, 0), (1, 1))
          * jax.lax.slice(u, (0, _D), (_L, 2 * _D)))

    d1 = _rms(h3)
    A1, bmB = _routing(d1, m2, dWq_ref[1], dbq_ref[1], dWk_ref[1], dbk_ref[1])
    u2 = _upsample2(z_ref, sc_ref, d1, e0_ref[0], A1, bmB)
    out_ref[0] = (jax.lax.slice(u2, (0, 0), (_L, _D))
                  + jax.lax.slice(rw, (0, 1), (1, 2))
                  * jax.lax.slice(u2, (0, _D), (_L, 2 * _D)))


def _dec_call(h2, m2_f, e1, e0, dWq, dbq, dWk, dbk, rw):
    B = h2.shape[0]
    nl = dWq.shape[0]
    full = lambda shape: pl.BlockSpec(shape, lambda b: (0,) * len(shape))
    return pl.pallas_call(
        _dec_body,
        grid=(B,),
        in_specs=[
            pl.BlockSpec((1, _L, _D), lambda b: (b, 0, 0)),
            pl.BlockSpec((1, 1, _L), lambda b: (b, 0, 0)),
            pl.BlockSpec((1, _L, _D), lambda b: (b, 0, 0)),
            pl.BlockSpec((1, _L, _D), lambda b: (b, 0, 0)),
            full((nl, _D, _D)), full((nl, 1, _D)),
            full((nl, _D, _D)), full((nl, 1, _D)),
            full((1, 1, nl)),
        ],
        out_specs=pl.BlockSpec((1, _L, _D), lambda b: (b, 0, 0)),
        out_shape=jax.ShapeDtypeStruct((B, _L, _D), jnp.float32),
        scratch_shapes=[pltpu.VMEM((_L, 2 * _D), jnp.float32),
                        pltpu.VMEM((1, _L), jnp.float32)],
    )(h2, m2_f, e1, e0, dWq, dbq.reshape(nl, 1, _D), dWk,
      dbk.reshape(nl, 1, _D), rw.astype(jnp.float32).reshape(1, 1, nl))


def kernel(hidden_states, mask, enc_Wq, enc_bq, enc_Wk, enc_bk,
           dec_Wq, dec_bq, dec_Wk, dec_bk, residual_weights):
    B = hidden_states.shape[0]
    m0 = mask.astype(jnp.float32).reshape(B, 1, _L)
    e0, ord0, m1 = _enc_call(hidden_states, m0,
                             enc_Wq[0], enc_bq[0], enc_Wk[0], enc_bk[0])
    h1 = _gather_rows(e0, ord0)
    e1, ord1, m2 = _enc_call(h1, m1,
                             enc_Wq[1], enc_bq[1], enc_Wk[1], enc_bk[1])
    h2 = _gather_rows(e1, ord1)
    return _dec_call(h2, m2, e1, e0, dec_Wq, dec_bq, dec_Wk, dec_bk,
                     residual_weights)
